# baseline (speedup 1.0000x reference)
import functools

import jax
import jax.numpy as jnp
from jax.experimental import pallas as pl
import jax.experimental.pallas.tpu as pltpu

_K = 32
_NEG_DIAG = -1e9
_NEG_TAKEN = -3e9
_EPR = 8


def _knn_kernel(a_ref, b_ref, scores_ref, idx_ref, s_ref, col_ref,
                *, rblk, seq, k, epr):
    i = pl.program_id(1)
    b_id = pl.program_id(0)

    @pl.when((b_id == 0) & (i == 0))
    def _():
        col_ref[...] = jax.lax.broadcasted_iota(
            jnp.int32, (seq, rblk), 0).astype(jnp.float32)

    a = a_ref[0]
    b = b_ref[0]

    st = jax.lax.dot_general(b, a, (((1,), (1,)), ((), ())),
                             preferred_element_type=jnp.float32)

    cnd = jax.lax.broadcasted_iota(jnp.int32, (seq, rblk), 0)
    row_g = i * rblk + jax.lax.broadcasted_iota(jnp.int32, (seq, rblk), 1)
    s_ref[...] = jnp.where(cnd == row_g, _NEG_DIAG, st)

    krow = jax.lax.broadcasted_iota(jnp.int32, (k, rblk), 0)

    def body(it, carry):
        vals, idxs = carry
        s = s_ref[...]
        colf = col_ref[...]
        for e in range(epr):
            kk = it * epr + e
            m = jnp.max(s, axis=0)
            cand = jnp.where(s >= m[None, :], colf, 3.0e9)
            posf = jnp.min(cand, axis=0)
            s = jnp.where(cand == posf[None, :], _NEG_TAKEN, s)
            pos = posf.astype(jnp.int32)
            sel = krow == kk
            vals = jnp.where(sel, m[None, :], vals)
            idxs = jnp.where(sel, pos[None, :], idxs)
        s_ref[...] = s
        return vals, idxs

    vals0 = jnp.full((k, rblk), 0.0, jnp.float32)
    idxs0 = jnp.full((k, rblk), 0, jnp.int32)
    vals, idxs = jax.lax.fori_loop(0, k // epr, body, (vals0, idxs0))
    scores_ref[0] = vals
    idx_ref[0] = idxs


@jax.jit
def kernel(embeddings):
    batch, seq, d = embeddings.shape
    k = min(_K, seq - 1)
    rblk = min(512, seq)
    nblk = seq // rblk
    epr = _EPR if k % _EPR == 0 else 1

    emb_n = embeddings / (
        jnp.linalg.norm(embeddings, axis=-1, keepdims=True) + 1e-08)

    kfn = functools.partial(_knn_kernel, rblk=rblk, seq=seq, k=k, epr=epr)
    scores_t, idxs_t = pl.pallas_call(
        kfn,
        grid=(batch, nblk),
        in_specs=[
            pl.BlockSpec((1, rblk, d), lambda b, i: (b, i, 0)),
            pl.BlockSpec((1, seq, d), lambda b, i: (b, 0, 0)),
        ],
        out_specs=[
            pl.BlockSpec((1, k, rblk), lambda b, i: (b, 0, i)),
            pl.BlockSpec((1, k, rblk), lambda b, i: (b, 0, i)),
        ],
        out_shape=[
            jax.ShapeDtypeStruct((batch, k, seq), jnp.float32),
            jax.ShapeDtypeStruct((batch, k, seq), jnp.int32),
        ],
        scratch_shapes=[pltpu.VMEM((seq, rblk), jnp.float32),
                        pltpu.VMEM((seq, rblk), jnp.float32)],
    )(emb_n, emb_n)

    scores = jnp.transpose(scores_t, (0, 2, 1))
    idxs = jnp.transpose(idxs_t, (0, 2, 1))

    if k < _K:
        pad = _K - k
        scores = jnp.concatenate(
            [scores, jnp.zeros((batch, seq, pad), scores.dtype)], axis=-1)
        idxs = jnp.concatenate(
            [idxs, jnp.zeros((batch, seq, pad), idxs.dtype)], axis=-1)
    half = _K // 2
    return (scores, idxs.astype(jnp.int64), scores[..., :half],
            -scores[..., half:])
